# Initial kernel scaffold; baseline (speedup 1.0000x reference)
import jax
import jax.numpy as jnp
from jax.experimental import pallas as pl


def kernel(x, edge_index, edge_weights, W1, b1, W2, b2):
    raise NotImplementedError("write your pallas kernel here")


# speedup vs baseline: 6.9609x; 6.9609x over previous
import functools

import jax
import jax.numpy as jnp
from jax import lax
from jax.experimental import pallas as pl
from jax.experimental.pallas import tpu as pltpu
from jax.experimental.pallas import tpu_sc as plsc

_NC = 2
_NS = 16
_NW = _NC * _NS
_L = 16
_CH = 80


def _sc_mesh():
  return plsc.VectorSubcoreMesh(
      core_axis_name="c", subcore_axis_name="s",
      num_cores=_NC, num_subcores=_NS)


def _stripe(n):
  ps = (-(-n // _NS) + 7) // 8 * 8
  return ps, ps * _NS


def _deg_kernel(dst, w, n, d):
  e = dst.shape[0]
  per_w = e // _NW
  n_chunks = per_w // _CH
  ps, np_ = _stripe(n)
  nk = d // _L

  @functools.partial(
      pl.kernel,
      out_type=jax.ShapeDtypeStruct((_NC, np_, d), jnp.float32),
      mesh=_sc_mesh(),
      scratch_types=[
          pltpu.VMEM((_CH,), jnp.int32),
          pltpu.VMEM((_CH,), jnp.float32),
          pltpu.VMEM((_CH, d), jnp.float32),
          pltpu.VMEM((8, d), jnp.float32),
          pltpu.VMEM_SHARED((np_, d), jnp.float32),
      ],
      compiler_params=pltpu.CompilerParams(needs_layout_passes=False),
  )
  def body(dst_hbm, w_hbm, degp_hbm, dstv, wv, wrows, zbuf, acc_s):
    c = lax.axis_index("c")
    s = lax.axis_index("s")
    wid = s * _NC + c
    off = s * ps
    lane0 = jnp.where(lax.iota(jnp.int32, _L) == 0,
                      jnp.float32(1.0), jnp.float32(0.0))

    for r in range(8):
      for k in range(nk):
        zbuf[r, pl.ds(k * _L, _L)] = jnp.zeros((_L,), jnp.float32)

    @pl.loop(0, _CH)
    def _zrows(ei):
      for k in range(nk):
        wrows[ei, pl.ds(k * _L, _L)] = jnp.zeros((_L,), jnp.float32)

    @pl.loop(0, ps // 8)
    def _zero(j):
      pltpu.sync_copy(zbuf, acc_s.at[pl.ds(off + j * 8, 8)])

    plsc.subcore_barrier()

    @pl.loop(0, n_chunks)
    def _chunks(ci):
      base = wid * per_w + ci * _CH
      pltpu.sync_copy(dst_hbm.at[pl.ds(base, _CH)], dstv)
      pltpu.sync_copy(w_hbm.at[pl.ds(base, _CH)], wv)

      @pl.loop(0, _CH, unroll=8)
      def _build(ei):
        ws = plsc.load_gather(wv, [jnp.full((_L,), ei, jnp.int32)])
        wrows[ei, pl.ds(0, _L)] = ws * lane0

      pltpu.sync_copy(wrows, acc_s.at[dstv], add=True)

    plsc.subcore_barrier()
    pltpu.sync_copy(acc_s.at[pl.ds(off, ps)], degp_hbm.at[c, pl.ds(off, ps)])

  return body(dst, w)


def _agg_kernel(src, dst, w, h, n):
  e = src.shape[0]
  d = h.shape[1]
  nk = d // _L
  per_w = e // _NW
  n_chunks = per_w // _CH
  ps, np_ = _stripe(n)

  @functools.partial(
      pl.kernel,
      out_type=jax.ShapeDtypeStruct((_NC, np_, d), jnp.float32),
      mesh=_sc_mesh(),
      scratch_types=[
          pltpu.VMEM((_CH,), jnp.int32),
          pltpu.VMEM((_CH,), jnp.int32),
          pltpu.VMEM((_CH,), jnp.float32),
          pltpu.VMEM((_CH, d), jnp.float32),
          pltpu.VMEM((8, d), jnp.float32),
          pltpu.VMEM_SHARED((np_, d), jnp.float32),
          pltpu.SemaphoreType.DMA,
      ],
      compiler_params=pltpu.CompilerParams(needs_layout_passes=False),
  )
  def body(src_hbm, dst_hbm, w_hbm, h_hbm, out_hbm,
           srcv, dstv, wv, rows, zbuf, acc_s, sem):
    c = lax.axis_index("c")
    s = lax.axis_index("s")
    wid = s * _NC + c
    off = s * ps

    for r in range(8):
      for k in range(nk):
        zbuf[r, pl.ds(k * _L, _L)] = jnp.zeros((_L,), jnp.float32)

    @pl.loop(0, ps // 8)
    def _zero(j):
      pltpu.sync_copy(zbuf, acc_s.at[pl.ds(off + j * 8, 8)])

    plsc.subcore_barrier()

    @pl.loop(0, n_chunks)
    def _chunks(ci):
      base = wid * per_w + ci * _CH
      pltpu.sync_copy(src_hbm.at[pl.ds(base, _CH)], srcv)
      pltpu.sync_copy(dst_hbm.at[pl.ds(base, _CH)], dstv)
      pltpu.sync_copy(w_hbm.at[pl.ds(base, _CH)], wv)
      pltpu.async_copy(h_hbm.at[srcv], rows, sem).wait()

      @pl.loop(0, _CH, unroll=8)
      def _scale(ei):
        ws = plsc.load_gather(wv, [jnp.full((_L,), ei, jnp.int32)])
        for k in range(nk):
          sl = pl.ds(k * _L, _L)
          rows[ei, sl] = rows[ei, sl] * ws

      pltpu.sync_copy(rows, acc_s.at[dstv], add=True)

    plsc.subcore_barrier()
    pltpu.sync_copy(acc_s.at[pl.ds(off, ps)], out_hbm.at[c, pl.ds(off, ps)])

  return body(src, dst, w, h)


def _tc_first(x, w1, degp):
  n, d = x.shape

  def body(x_ref, w_ref, degp_ref, h_ref, dinv_ref):
    deg = (degp_ref[0] + degp_ref[1])[:n, 0:1]
    dinv = jnp.where(deg > 0, lax.rsqrt(deg), 0.0)
    dinv_ref[...] = dinv
    h = jnp.dot(x_ref[...], w_ref[...], preferred_element_type=jnp.float32)
    h_ref[...] = h * dinv

  return pl.pallas_call(
      body,
      out_shape=[jax.ShapeDtypeStruct((n, d), jnp.float32),
                 jax.ShapeDtypeStruct((n, 1), jnp.float32)],
  )(x, w1, degp)


def _tc_mid(accp, dinv, b, w2):
  n = dinv.shape[0]
  d = accp.shape[2]

  def body(accp_ref, dinv_ref, b_ref, w_ref, h_ref):
    dv = dinv_ref[...]
    x2 = (accp_ref[0] + accp_ref[1])[:n] * dv + b_ref[...]
    h = jnp.dot(x2, w_ref[...], preferred_element_type=jnp.float32)
    h_ref[...] = h * dv

  return pl.pallas_call(
      body,
      out_shape=jax.ShapeDtypeStruct((n, d), jnp.float32),
  )(accp, dinv, b, w2)


def _tc_last(accp, dinv, b):
  n = dinv.shape[0]
  d = accp.shape[2]

  def body(accp_ref, dinv_ref, b_ref, o_ref):
    o = (accp_ref[0] + accp_ref[1])[:n] * dinv_ref[...] + b_ref[...]
    o_ref[...] = jnp.maximum(o, 0.0)

  return pl.pallas_call(
      body,
      out_shape=jax.ShapeDtypeStruct((n, d), jnp.float32),
  )(accp, dinv, b)


def kernel(x, edge_index, edge_weights, W1, b1, W2, b2):
  n, d = x.shape
  e = edge_weights.shape[0]
  assert e % _NW == 0 and (e // _NW) % _CH == 0 and d % _L == 0
  src = edge_index[0]
  dst = edge_index[1]

  degp = _deg_kernel(dst, edge_weights, n, d)
  h1, dinv = _tc_first(x, W1, degp)
  acc1 = _agg_kernel(src, dst, edge_weights, h1, n)
  h2 = _tc_mid(acc1, dinv, b1, W2)
  acc2 = _agg_kernel(src, dst, edge_weights, h2, n)
  return _tc_last(acc2, dinv, b2)
